# baseline (speedup 1.0000x reference)
import functools

import jax
import jax.numpy as jnp
from jax import lax
from jax.experimental import pallas as pl
from jax.experimental.pallas import tpu as pltpu
from jax.experimental.pallas import tpu_sc as plsc

N = 10000
E = 320000
D = 128
NPAD = 10240
NW = 32
CHUNK = 128
NCH = 80
HALF = NCH // 2
EPW = NCH * CHUNK
EPAD = NW * EPW
ROWS_PER_TILE = NPAD // 16
PAD_ROW = N

_mesh = plsc.VectorSubcoreMesh(core_axis_name="c", subcore_axis_name="s")


@functools.partial(
    pl.kernel,
    out_type=jax.ShapeDtypeStruct((2, NPAD, D), jnp.float32),
    mesh=_mesh,
    scratch_types=[
        pltpu.VMEM((HALF, CHUNK), jnp.int32),
        pltpu.VMEM((HALF, CHUNK), jnp.int32),
        pltpu.VMEM((CHUNK, D), jnp.float32),
        pltpu.VMEM((CHUNK, D), jnp.float32),
        pltpu.VMEM_SHARED((NPAD, D), jnp.float32),
        pltpu.SemaphoreType.DMA,
        pltpu.SemaphoreType.DMA,
    ],
)
def _sc_aggregate(xp_hbm, srcw_hbm, dstw_hbm, out_hbm, sidx, didx, rows0, rows1,
                  acc, sem0, sem1):
    c = lax.axis_index("c")
    s = lax.axis_index("s")
    w = s * 2 + c

    def zrow(i, _):
        for j in range(D // 16):
            rows0[i, pl.ds(j * 16, 16)] = jnp.zeros((16,), jnp.float32)
        return 0

    lax.fori_loop(0, CHUNK, zrow, 0)
    for k in range(ROWS_PER_TILE // CHUNK):
        pltpu.async_copy(
            rows0, acc.at[pl.ds(s * ROWS_PER_TILE + k * CHUNK, CHUNK)], sem0
        )
    for k in range(ROWS_PER_TILE // CHUNK):
        pltpu.make_async_copy(
            rows0, acc.at[pl.ds(s * ROWS_PER_TILE + k * CHUNK, CHUNK)], sem0
        ).wait()
    plsc.subcore_barrier()

    for h in range(2):
        pltpu.sync_copy(srcw_hbm.at[w, pl.ds(h * HALF, HALF)], sidx)
        pltpu.sync_copy(dstw_hbm.at[w, pl.ds(h * HALF, HALF)], didx)
        pltpu.async_copy(xp_hbm.at[sidx.at[0]], rows0, sem0)

        def body(g, _):
            j0 = 2 * g
            pltpu.async_copy(xp_hbm.at[sidx.at[j0 + 1]], rows1, sem1)
            pltpu.make_async_copy(xp_hbm.at[sidx.at[j0]], rows0, sem0).wait()
            pltpu.sync_copy(rows0, acc.at[didx.at[j0]], add=True)

            @pl.when(j0 + 2 < HALF)
            def _():
                pltpu.async_copy(xp_hbm.at[sidx.at[j0 + 2]], rows0, sem0)

            pltpu.make_async_copy(xp_hbm.at[sidx.at[j0 + 1]], rows1, sem1).wait()
            pltpu.sync_copy(rows1, acc.at[didx.at[j0 + 1]], add=True)
            return 0

        lax.fori_loop(0, HALF // 2, body, 0)
    plsc.subcore_barrier()

    pltpu.sync_copy(
        acc.at[pl.ds(s * ROWS_PER_TILE, ROWS_PER_TILE)],
        out_hbm.at[c, pl.ds(s * ROWS_PER_TILE, ROWS_PER_TILE)],
    )


def _mm_body(x_ref, w_ref, b_ref, o_ref):
    o_ref[...] = (
        lax.dot_general(
            x_ref[...], w_ref[...], (((1,), (1,)), ((), ())),
            preferred_element_type=jnp.float32,
        )
        + b_ref[...]
    )


def _mid_body(xp_ref, p0_ref, p1_ref, c_ref, w_ref, b_ref, o_ref):
    t = (xp_ref[...] + p0_ref[0] + p1_ref[0]) * (-jnp.abs(c_ref[0]))
    h = jnp.maximum(t, 0.0)
    o_ref[...] = (
        lax.dot_general(
            h, w_ref[...], (((1,), (1,)), ((), ())),
            preferred_element_type=jnp.float32,
        )
        + b_ref[...]
    )


def _final_body(xp_ref, q0_ref, q1_ref, c_ref, o_ref):
    o = (xp_ref[...] + q0_ref[0] + q1_ref[0]) * (-jnp.abs(c_ref[0]))
    m = jnp.max(o, axis=1, keepdims=True)
    lse = jnp.log(jnp.sum(jnp.exp(o - m), axis=1, keepdims=True)) + m
    o_ref[...] = o - lse


_BLK = 1024


def _mm(x, W, b):
    return pl.pallas_call(
        _mm_body,
        grid=(NPAD // _BLK,),
        in_specs=[
            pl.BlockSpec((_BLK, D), lambda i: (i, 0)),
            pl.BlockSpec((D, D), lambda i: (0, 0)),
            pl.BlockSpec((1, D), lambda i: (0, 0)),
        ],
        out_specs=pl.BlockSpec((_BLK, D), lambda i: (i, 0)),
        out_shape=jax.ShapeDtypeStruct((NPAD, D), jnp.float32),
    )(x, W, b.reshape(1, D))


def _mid(xp, p, c, W, b):
    return pl.pallas_call(
        _mid_body,
        grid=(NPAD // _BLK,),
        in_specs=[
            pl.BlockSpec((_BLK, D), lambda i: (i, 0)),
            pl.BlockSpec((1, _BLK, D), lambda i: (0, i, 0)),
            pl.BlockSpec((1, _BLK, D), lambda i: (1, i, 0)),
            pl.BlockSpec(memory_space=pltpu.SMEM),
            pl.BlockSpec((D, D), lambda i: (0, 0)),
            pl.BlockSpec((1, D), lambda i: (0, 0)),
        ],
        out_specs=pl.BlockSpec((_BLK, D), lambda i: (i, 0)),
        out_shape=jax.ShapeDtypeStruct((NPAD, D), jnp.float32),
    )(xp, p, p, c, W, b.reshape(1, D))


_FBLK = 1000


def _final(xp, q, c):
    return pl.pallas_call(
        _final_body,
        grid=(N // _FBLK,),
        in_specs=[
            pl.BlockSpec((_FBLK, D), lambda i: (i, 0)),
            pl.BlockSpec((1, _FBLK, D), lambda i: (0, i, 0)),
            pl.BlockSpec((1, _FBLK, D), lambda i: (1, i, 0)),
            pl.BlockSpec(memory_space=pltpu.SMEM),
        ],
        out_specs=pl.BlockSpec((_FBLK, D), lambda i: (i, 0)),
        out_shape=jax.ShapeDtypeStruct((N, D), jnp.float32),
    )(xp, q, q, c)


def kernel(x, edge_index, W1, b1, Wa1, ba1, c1, W2, b2, Wa2, ba2, c2):
    del Wa1, ba1, Wa2, ba2
    src = edge_index[0].astype(jnp.int32)
    dst = edge_index[1].astype(jnp.int32)
    pad = PAD_ROW + (jnp.arange(EPAD - E, dtype=jnp.int32) % (NPAD - N))
    srcw = jnp.concatenate([src, pad]).reshape(NW, NCH, CHUNK)
    dstw = jnp.concatenate([dst, pad]).reshape(NW, NCH, CHUNK)

    xp1 = _mm(x, W1, b1)
    p = _sc_aggregate(xp1, srcw, dstw)
    xp2 = _mid(xp1, p, c1, W2, b2)
    q = _sc_aggregate(xp2, srcw, dstw)
    return _final(xp2, q, c2)


# speedup vs baseline: 1.0237x; 1.0237x over previous
import functools

import jax
import jax.numpy as jnp
from jax import lax
from jax.experimental import pallas as pl
from jax.experimental.pallas import tpu as pltpu
from jax.experimental.pallas import tpu_sc as plsc

N = 10000
E = 320000
D = 128
NPAD = 10240
NW = 32
CHUNK = 128
NCH = 80
HALF = NCH // 2
EPW = NCH * CHUNK
EPAD = NW * EPW
ROWS_PER_TILE = NPAD // 16
PAD_ROW = N

_mesh = plsc.VectorSubcoreMesh(core_axis_name="c", subcore_axis_name="s")


@functools.partial(
    pl.kernel,
    out_type=jax.ShapeDtypeStruct((2, NPAD, D), jnp.float32),
    mesh=_mesh,
    scratch_types=[
        pltpu.VMEM((HALF, CHUNK), jnp.int32),
        pltpu.VMEM((HALF, CHUNK), jnp.int32),
        pltpu.VMEM((CHUNK, D), jnp.float32),
        pltpu.VMEM((CHUNK, D), jnp.float32),
        pltpu.VMEM_SHARED((NPAD, D), jnp.float32),
        pltpu.SemaphoreType.DMA,
        pltpu.SemaphoreType.DMA,
    ],
)
def _sc_aggregate(xp_hbm, srcw_hbm, dstw_hbm, out_hbm, sidx, didx, rows0, rows1,
                  acc, sem0, sem1):
    c = lax.axis_index("c")
    s = lax.axis_index("s")
    w = s * 2 + c

    def zrow(i, _):
        for j in range(D // 16):
            rows0[i, pl.ds(j * 16, 16)] = jnp.zeros((16,), jnp.float32)
        return 0

    lax.fori_loop(0, CHUNK, zrow, 0)
    for k in range(ROWS_PER_TILE // CHUNK):
        pltpu.async_copy(
            rows0, acc.at[pl.ds(s * ROWS_PER_TILE + k * CHUNK, CHUNK)], sem0
        )
    for k in range(ROWS_PER_TILE // CHUNK):
        pltpu.make_async_copy(
            rows0, acc.at[pl.ds(s * ROWS_PER_TILE + k * CHUNK, CHUNK)], sem0
        ).wait()
    plsc.subcore_barrier()

    for h in range(2):
        pltpu.sync_copy(srcw_hbm.at[w, pl.ds(h * HALF, HALF)], sidx)
        pltpu.sync_copy(dstw_hbm.at[w, pl.ds(h * HALF, HALF)], didx)
        pltpu.async_copy(xp_hbm.at[sidx.at[0]], rows0, sem0)

        def body(g, _):
            j0 = 2 * g
            pltpu.async_copy(xp_hbm.at[sidx.at[j0 + 1]], rows1, sem1)
            pltpu.make_async_copy(xp_hbm.at[sidx.at[j0]], rows0, sem0).wait()
            pltpu.sync_copy(rows0, acc.at[didx.at[j0]], add=True)

            @pl.when(j0 + 2 < HALF)
            def _():
                pltpu.async_copy(xp_hbm.at[sidx.at[j0 + 2]], rows0, sem0)

            pltpu.make_async_copy(xp_hbm.at[sidx.at[j0 + 1]], rows1, sem1).wait()
            pltpu.sync_copy(rows1, acc.at[didx.at[j0 + 1]], add=True)
            return 0

        lax.fori_loop(0, HALF // 2, body, 0)
    plsc.subcore_barrier()

    pltpu.sync_copy(
        acc.at[pl.ds(s * ROWS_PER_TILE, ROWS_PER_TILE)],
        out_hbm.at[c, pl.ds(s * ROWS_PER_TILE, ROWS_PER_TILE)],
    )


def _mm_body(x_ref, w_ref, b_ref, o_ref):
    o_ref[...] = (
        lax.dot_general(
            x_ref[...], w_ref[...], (((1,), (1,)), ((), ())),
            preferred_element_type=jnp.float32,
        )
        + b_ref[...]
    )


def _mid_body(xp_ref, p0_ref, p1_ref, c_ref, w_ref, b_ref, o_ref):
    t = (xp_ref[...] + p0_ref[0] + p1_ref[0]) * (-jnp.abs(c_ref[0]))
    h = jnp.maximum(t, 0.0)
    o_ref[...] = (
        lax.dot_general(
            h, w_ref[...], (((1,), (1,)), ((), ())),
            preferred_element_type=jnp.float32,
        )
        + b_ref[...]
    )


def _final_body(xp_ref, q0_ref, q1_ref, c_ref, o_ref):
    o = (xp_ref[...] + q0_ref[0] + q1_ref[0]) * (-jnp.abs(c_ref[0]))
    m = jnp.max(o, axis=1, keepdims=True)
    lse = jnp.log(jnp.sum(jnp.exp(o - m), axis=1, keepdims=True)) + m
    o_ref[...] = o - lse


_BLK = 2048


def _mm(x, W, b):
    return pl.pallas_call(
        _mm_body,
        grid=(NPAD // _BLK,),
        in_specs=[
            pl.BlockSpec((_BLK, D), lambda i: (i, 0)),
            pl.BlockSpec((D, D), lambda i: (0, 0)),
            pl.BlockSpec((1, D), lambda i: (0, 0)),
        ],
        out_specs=pl.BlockSpec((_BLK, D), lambda i: (i, 0)),
        out_shape=jax.ShapeDtypeStruct((NPAD, D), jnp.float32),
    )(x, W, b.reshape(1, D))


def _mid(xp, p, c, W, b):
    return pl.pallas_call(
        _mid_body,
        grid=(NPAD // _BLK,),
        in_specs=[
            pl.BlockSpec((_BLK, D), lambda i: (i, 0)),
            pl.BlockSpec((1, _BLK, D), lambda i: (0, i, 0)),
            pl.BlockSpec((1, _BLK, D), lambda i: (1, i, 0)),
            pl.BlockSpec(memory_space=pltpu.SMEM),
            pl.BlockSpec((D, D), lambda i: (0, 0)),
            pl.BlockSpec((1, D), lambda i: (0, 0)),
        ],
        out_specs=pl.BlockSpec((_BLK, D), lambda i: (i, 0)),
        out_shape=jax.ShapeDtypeStruct((NPAD, D), jnp.float32),
    )(xp, p, p, c, W, b.reshape(1, D))


_FBLK = 2000


def _final(xp, q, c):
    return pl.pallas_call(
        _final_body,
        grid=(N // _FBLK,),
        in_specs=[
            pl.BlockSpec((_FBLK, D), lambda i: (i, 0)),
            pl.BlockSpec((1, _FBLK, D), lambda i: (0, i, 0)),
            pl.BlockSpec((1, _FBLK, D), lambda i: (1, i, 0)),
            pl.BlockSpec(memory_space=pltpu.SMEM),
        ],
        out_specs=pl.BlockSpec((_FBLK, D), lambda i: (i, 0)),
        out_shape=jax.ShapeDtypeStruct((N, D), jnp.float32),
    )(xp, q, q, c)


def kernel(x, edge_index, W1, b1, Wa1, ba1, c1, W2, b2, Wa2, ba2, c2):
    del Wa1, ba1, Wa2, ba2
    src = edge_index[0].astype(jnp.int32)
    dst = edge_index[1].astype(jnp.int32)
    pad = PAD_ROW + (jnp.arange(EPAD - E, dtype=jnp.int32) % (NPAD - N))
    srcw = jnp.concatenate([src, pad]).reshape(NW, NCH, CHUNK)
    dstw = jnp.concatenate([dst, pad]).reshape(NW, NCH, CHUNK)

    xp1 = _mm(x, W1, b1)
    p = _sc_aggregate(xp1, srcw, dstw)
    xp2 = _mid(xp1, p, c1, W2, b2)
    q = _sc_aggregate(xp2, srcw, dstw)
    return _final(xp2, q, c2)


# speedup vs baseline: 1.0325x; 1.0086x over previous
import functools

import jax
import jax.numpy as jnp
from jax import lax
from jax.experimental import pallas as pl
from jax.experimental.pallas import tpu as pltpu
from jax.experimental.pallas import tpu_sc as plsc

N = 10000
E = 320000
D = 128
NPAD = 10240
NW = 32
CHUNK = 128
NCH = 80
HALF = NCH // 2
EPW = NCH * CHUNK
EPAD = NW * EPW
ROWS_PER_TILE = NPAD // 16
PAD_ROW = N

_mesh = plsc.VectorSubcoreMesh(core_axis_name="c", subcore_axis_name="s")


@functools.partial(
    pl.kernel,
    out_type=jax.ShapeDtypeStruct((2, NPAD, D), jnp.float32),
    mesh=_mesh,
    scratch_types=[
        pltpu.VMEM((HALF, CHUNK), jnp.int32),
        pltpu.VMEM((HALF, CHUNK), jnp.int32),
        pltpu.VMEM((CHUNK, D), jnp.float32),
        pltpu.VMEM((CHUNK, D), jnp.float32),
        pltpu.VMEM_SHARED((NPAD, D), jnp.float32),
        pltpu.SemaphoreType.DMA,
        pltpu.SemaphoreType.DMA,
    ],
)
def _sc_aggregate(xp_hbm, srcw_hbm, dstw_hbm, out_hbm, sidx, didx, rows0, rows1,
                  acc, sem0, sem1):
    c = lax.axis_index("c")
    s = lax.axis_index("s")
    w = s * 2 + c

    def zrow(i, _):
        for j in range(D // 16):
            rows0[i, pl.ds(j * 16, 16)] = jnp.zeros((16,), jnp.float32)
        return 0

    lax.fori_loop(0, CHUNK, zrow, 0)
    for k in range(ROWS_PER_TILE // CHUNK):
        pltpu.async_copy(
            rows0, acc.at[pl.ds(s * ROWS_PER_TILE + k * CHUNK, CHUNK)], sem0
        )
    for k in range(ROWS_PER_TILE // CHUNK):
        pltpu.make_async_copy(
            rows0, acc.at[pl.ds(s * ROWS_PER_TILE + k * CHUNK, CHUNK)], sem0
        ).wait()
    plsc.subcore_barrier()

    for h in range(2):
        pltpu.sync_copy(srcw_hbm.at[w, pl.ds(h * HALF, HALF)], sidx)
        pltpu.sync_copy(dstw_hbm.at[w, pl.ds(h * HALF, HALF)], didx)
        pltpu.async_copy(xp_hbm.at[sidx.at[0]], rows0, sem0)

        def body(g, _):
            j0 = 2 * g
            pltpu.async_copy(xp_hbm.at[sidx.at[j0 + 1]], rows1, sem1)
            pltpu.make_async_copy(xp_hbm.at[sidx.at[j0]], rows0, sem0).wait()
            pltpu.sync_copy(rows0, acc.at[didx.at[j0]], add=True)

            @pl.when(j0 + 2 < HALF)
            def _():
                pltpu.async_copy(xp_hbm.at[sidx.at[j0 + 2]], rows0, sem0)

            pltpu.make_async_copy(xp_hbm.at[sidx.at[j0 + 1]], rows1, sem1).wait()
            pltpu.sync_copy(rows1, acc.at[didx.at[j0 + 1]], add=True)
            return 0

        lax.fori_loop(0, HALF // 2, body, 0)
    plsc.subcore_barrier()

    pltpu.sync_copy(
        acc.at[pl.ds(s * ROWS_PER_TILE, ROWS_PER_TILE)],
        out_hbm.at[c, pl.ds(s * ROWS_PER_TILE, ROWS_PER_TILE)],
    )


def _mm_body(x_ref, w_ref, b_ref, o_ref):
    o_ref[...] = (
        lax.dot_general(
            x_ref[...], w_ref[...], (((1,), (1,)), ((), ())),
            preferred_element_type=jnp.float32,
        )
        + b_ref[...]
    )


def _mid_body(xp_ref, p0_ref, p1_ref, c_ref, w_ref, b_ref, o_ref):
    t = (xp_ref[...] + p0_ref[0] + p1_ref[0]) * (-jnp.abs(c_ref[0]))
    h = jnp.maximum(t, 0.0)
    o_ref[...] = (
        lax.dot_general(
            h, w_ref[...], (((1,), (1,)), ((), ())),
            preferred_element_type=jnp.float32,
        )
        + b_ref[...]
    )


def _final_body(xp_ref, q0_ref, q1_ref, c_ref, o_ref):
    o = (xp_ref[...] + q0_ref[0] + q1_ref[0]) * (-jnp.abs(c_ref[0]))
    m = jnp.max(o, axis=1, keepdims=True)
    lse = jnp.log(jnp.sum(jnp.exp(o - m), axis=1, keepdims=True)) + m
    o_ref[...] = o - lse


_BLK = 2560


def _mm(x, W, b):
    return pl.pallas_call(
        _mm_body,
        grid=(NPAD // _BLK,),
        in_specs=[
            pl.BlockSpec((_BLK, D), lambda i: (i, 0)),
            pl.BlockSpec((D, D), lambda i: (0, 0)),
            pl.BlockSpec((1, D), lambda i: (0, 0)),
        ],
        out_specs=pl.BlockSpec((_BLK, D), lambda i: (i, 0)),
        out_shape=jax.ShapeDtypeStruct((NPAD, D), jnp.float32),
    )(x, W, b.reshape(1, D))


def _mid(xp, p, c, W, b):
    return pl.pallas_call(
        _mid_body,
        grid=(NPAD // _BLK,),
        in_specs=[
            pl.BlockSpec((_BLK, D), lambda i: (i, 0)),
            pl.BlockSpec((1, _BLK, D), lambda i: (0, i, 0)),
            pl.BlockSpec((1, _BLK, D), lambda i: (1, i, 0)),
            pl.BlockSpec(memory_space=pltpu.SMEM),
            pl.BlockSpec((D, D), lambda i: (0, 0)),
            pl.BlockSpec((1, D), lambda i: (0, 0)),
        ],
        out_specs=pl.BlockSpec((_BLK, D), lambda i: (i, 0)),
        out_shape=jax.ShapeDtypeStruct((NPAD, D), jnp.float32),
    )(xp, p, p, c, W, b.reshape(1, D))


_FBLK = 2000


def _final(xp, q, c):
    return pl.pallas_call(
        _final_body,
        grid=(N // _FBLK,),
        in_specs=[
            pl.BlockSpec((_FBLK, D), lambda i: (i, 0)),
            pl.BlockSpec((1, _FBLK, D), lambda i: (0, i, 0)),
            pl.BlockSpec((1, _FBLK, D), lambda i: (1, i, 0)),
            pl.BlockSpec(memory_space=pltpu.SMEM),
        ],
        out_specs=pl.BlockSpec((_FBLK, D), lambda i: (i, 0)),
        out_shape=jax.ShapeDtypeStruct((N, D), jnp.float32),
    )(xp, q, q, c)


def kernel(x, edge_index, W1, b1, Wa1, ba1, c1, W2, b2, Wa2, ba2, c2):
    del Wa1, ba1, Wa2, ba2
    src = edge_index[0].astype(jnp.int32)
    dst = edge_index[1].astype(jnp.int32)
    pad = PAD_ROW + (jnp.arange(EPAD - E, dtype=jnp.int32) % (NPAD - N))
    srcw = jnp.concatenate([src, pad]).reshape(NW, NCH, CHUNK)
    dstw = jnp.concatenate([dst, pad]).reshape(NW, NCH, CHUNK)

    xp1 = _mm(x, W1, b1)
    p = _sc_aggregate(xp1, srcw, dstw)
    xp2 = _mid(xp1, p, c1, W2, b2)
    q = _sc_aggregate(xp2, srcw, dstw)
    return _final(xp2, q, c2)
